# baseline (device time: 70752 ns/iter reference)
import jax
import jax.numpy as jnp
from jax import lax
from jax.experimental import pallas as pl
from jax.experimental.pallas import tpu as pltpu

N_DEV = 4
B, S, D = 2, 512, 2048
H, Dh, Dr = 16, 128, 32
HG = H // N_DEV
GC = HG * Dh
DCS = 512 // N_DEV
BS = B * S
F32 = jnp.float32
BF16 = jnp.bfloat16


def _mla_body(x_ref, wdkv_ref, wuk_ref, wuv_ref, wq_ref, wqr_ref,
              wkr_ref, wo_ref, out_ref,
              x2s, cf, wkg, wvg, qg, qrg, krb, kg, vg, scs, og, obuf, wos,
              g_send, g_recv, a_send, a_recv, wo_sems):
    me = lax.axis_index("i")

    srcs = [me, (me + 3) % N_DEV, (me + 1) % N_DEV, (me + 2) % N_DEV]

    def wo_dma(j):
        return pltpu.make_async_copy(
            wo_ref.at[pl.ds(srcs[j] * GC, GC), :], wos.at[j % 2],
            wo_sems.at[j % 2])

    with jax.named_scope("entry_barrier"):
        barrier = pltpu.get_barrier_semaphore()
        for k in range(1, N_DEV):
            pl.semaphore_signal(barrier, inc=1,
                                device_id=((me + k) % N_DEV,),
                                device_id_type=pl.DeviceIdType.MESH)
        pl.semaphore_wait(barrier, N_DEV - 1)

    dma0 = wo_dma(0)
    dma0.start()
    dma1 = wo_dma(1)
    dma1.start()

    with jax.named_scope("c_proj"):
        x2s[...] = x_ref[...].astype(BF16)
        x2 = x2s[...]
        cf[:, pl.ds(me * DCS, DCS)] = jnp.dot(
            x2, wdkv_ref[...], preferred_element_type=F32).astype(BF16)
        wkg[pl.ds(me * DCS, DCS), :] = wuk_ref[:, pl.ds(me * GC, GC)]
        wvg[pl.ds(me * DCS, DCS), :] = wuv_ref[:, pl.ds(me * GC, GC)]

    gather = {}
    for k in (1, 3, 2):
        tgt = (me + k) % N_DEV
        base = 3 * (N_DEV - 1 - k)
        trio = []
        for t, (src, dst) in enumerate([
            (cf.at[:, pl.ds(me * DCS, DCS)],
             cf.at[:, pl.ds(me * DCS, DCS)]),
            (wuk_ref.at[:, pl.ds(tgt * GC, GC)],
             wkg.at[pl.ds(me * DCS, DCS), :]),
            (wuv_ref.at[:, pl.ds(tgt * GC, GC)],
             wvg.at[pl.ds(me * DCS, DCS), :]),
        ]):
            rdma = pltpu.make_async_remote_copy(
                src_ref=src, dst_ref=dst,
                send_sem=g_send.at[base + t], recv_sem=g_recv.at[base + t],
                device_id=(tgt,), device_id_type=pl.DeviceIdType.MESH,
            )
            rdma.start()
            trio.append(rdma)
        gather[k] = trio

    with jax.named_scope("q_proj"):
        qg[...] = jnp.dot(x2, wq_ref[...].astype(BF16),
                          preferred_element_type=F32).astype(BF16)
        qrg[...] = jnp.dot(x2, wqr_ref[...].astype(BF16),
                           preferred_element_type=F32).astype(BF16)
        krb[...] = jnp.dot(x2, wkr_ref[...],
                           preferred_element_type=F32).astype(BF16)

    scale = (Dh + Dr) ** -0.5
    cdims = (((1,), (1,)), ((), ()))
    with jax.named_scope("qr_scores"):
        for h in range(HG):
            for b in range(B):
                rows = pl.ds(b * S, S)
                scs[h * B + b] = lax.dot_general(
                    qrg[rows, pl.ds(h * Dr, Dr)], krb[rows, :], cdims,
                    preferred_element_type=F32).astype(BF16)

    with jax.named_scope("kv_own"):
        own_rows = pl.ds(me * DCS, DCS)
        kg[...] = jnp.dot(cf[:, own_rows], wkg[own_rows, :],
                          preferred_element_type=F32).astype(BF16)
        vg[...] = jnp.dot(cf[:, own_rows], wvg[own_rows, :],
                          preferred_element_type=F32).astype(BF16)
    for k in (1, 3, 2):
        with jax.named_scope(f"gather_wait#k={k}"):
            for rdma in gather[k]:
                rdma.wait()
        with jax.named_scope(f"kv_acc#k={k}"):
            p = (me + N_DEV - k) % N_DEV
            prows = pl.ds(p * DCS, DCS)
            kg[...] = (kg[...] + jnp.dot(
                cf[:, prows], wkg[prows, :],
                preferred_element_type=F32).astype(BF16))
            vg[...] = (vg[...] + jnp.dot(
                cf[:, prows], wvg[prows, :],
                preferred_element_type=F32).astype(BF16))

    ag = [[None] * HG for _ in range(N_DEV - 1)]
    for h in range(HG):
        cols = pl.ds(h * Dh, Dh)
        with jax.named_scope(f"attn#h={h}"):
            for b in range(B):
                rows = pl.ds(b * S, S)
                q = qg[rows, cols]
                k_ = kg[rows, cols]
                sc = lax.dot_general(q, k_, cdims,
                                     preferred_element_type=F32)
                sc = (sc + scs[h * B + b].astype(F32)) * scale
                m = jnp.max(sc, axis=1, keepdims=True)
                e = jnp.exp(sc - m)
                p = (e / jnp.sum(e, axis=1, keepdims=True)).astype(BF16)
                og[rows, cols] = jnp.dot(
                    p, vg[rows, cols],
                    preferred_element_type=F32).astype(BF16)
        for k in (1, 3, 2):
            tgt = (me + k) % N_DEV
            slot = N_DEV - 1 - k
            rdma = pltpu.make_async_remote_copy(
                src_ref=og.at[:, cols],
                dst_ref=obuf.at[slot, :, cols],
                send_sem=a_send.at[slot * HG + h],
                recv_sem=a_recv.at[slot * HG + h],
                device_id=(tgt,), device_id_type=pl.DeviceIdType.MESH,
            )
            rdma.start()
            ag[slot][h] = rdma

    with jax.named_scope("wo_own"):
        dma0.wait()
        out_ref[...] = jnp.dot(og[...], wos[0].astype(BF16),
                               preferred_element_type=F32)
        dma2 = wo_dma(2)
        dma2.start()
    with jax.named_scope("ag_wait#j=1"):
        for h in range(HG):
            ag[2][h].wait()
        dma1.wait()
    with jax.named_scope("wo_blk#j=1"):
        out_ref[...] += jnp.dot(obuf[2], wos[1].astype(BF16),
                                preferred_element_type=F32)
        dma3 = wo_dma(3)
        dma3.start()
    with jax.named_scope("ag_wait#j=2"):
        for h in range(HG):
            ag[0][h].wait()
        dma2.wait()
    with jax.named_scope("wo_blk#j=2"):
        out_ref[...] += jnp.dot(obuf[0], wos[0].astype(BF16),
                                preferred_element_type=F32)
    with jax.named_scope("ag_wait#j=3"):
        for h in range(HG):
            ag[1][h].wait()
        dma3.wait()
    with jax.named_scope("wo_blk#j=3"):
        out_ref[...] += jnp.dot(obuf[1], wos[1].astype(BF16),
                                preferred_element_type=F32)


def kernel(x, Wdkv, Wuk, Wuv, Wq, Wqr, Wkr, Wo):
    f = BF16
    me = lax.axis_index("i")
    x2 = x.reshape(BS, D)
    Wq_g = lax.dynamic_slice(Wq, (0, me * GC), (D, GC))
    Wqr_g = lax.dynamic_slice(Wqr, (0, me * HG * Dr), (D, HG * Dr))
    out = pl.pallas_call(
        _mla_body,
        out_shape=jax.ShapeDtypeStruct((BS, D), F32),
        in_specs=[pl.BlockSpec(memory_space=pltpu.VMEM)] * 7
        + [pl.BlockSpec(memory_space=pltpu.MemorySpace.HBM)],
        out_specs=pl.BlockSpec(memory_space=pltpu.VMEM),
        scratch_shapes=[
            pltpu.VMEM((BS, D), f),
            pltpu.VMEM((BS, N_DEV * DCS), f),
            pltpu.VMEM((N_DEV * DCS, GC), f),
            pltpu.VMEM((N_DEV * DCS, GC), f),
            pltpu.VMEM((BS, GC), f),
            pltpu.VMEM((BS, HG * Dr), f),
            pltpu.VMEM((BS, Dr), f),
            pltpu.VMEM((BS, GC), f),
            pltpu.VMEM((BS, GC), f),
            pltpu.VMEM((HG * B, S, S), f),
            pltpu.VMEM((BS, GC), f),
            pltpu.VMEM((N_DEV - 1, BS, GC), f),
            pltpu.VMEM((2, GC, D), F32),
            pltpu.SemaphoreType.DMA((3 * (N_DEV - 1),)),
            pltpu.SemaphoreType.DMA((3 * (N_DEV - 1),)),
            pltpu.SemaphoreType.DMA((HG * (N_DEV - 1),)),
            pltpu.SemaphoreType.DMA((HG * (N_DEV - 1),)),
            pltpu.SemaphoreType.DMA((2,)),
        ],
        compiler_params=pltpu.CompilerParams(collective_id=0),
    )(x2, Wdkv.astype(f), Wuk.astype(f), Wuv.astype(f), Wq_g, Wqr_g,
      Wkr.astype(f), Wo)
    return out.reshape(B, S, D)


# device time: 64843 ns/iter; 1.0911x vs baseline; 1.0911x over previous
import jax
import jax.numpy as jnp
from jax import lax
from jax.experimental import pallas as pl
from jax.experimental.pallas import tpu as pltpu

N_DEV = 4
B, S, D = 2, 512, 2048
H, Dh, Dr = 16, 128, 32
HG = H // N_DEV
GC = HG * Dh
DCS = 512 // N_DEV
BS = B * S
F32 = jnp.float32
BF16 = jnp.bfloat16


def _mla_body(x2_ref, wdkv_ref, wuk_ref, wuv_ref, wq_ref, wqr_ref,
              wkr_ref, wo_ref, out_ref,
              cf, wkg, wvg, qg, qrg, krb, kg, vg, scs, og, obuf, wos,
              g_send, g_recv, a_send, a_recv, wo_sems):
    me = lax.axis_index("i")

    with jax.named_scope("entry_barrier"):
        barrier = pltpu.get_barrier_semaphore()
        for k in range(1, N_DEV):
            pl.semaphore_signal(barrier, inc=1,
                                device_id=((me + k) % N_DEV,),
                                device_id_type=pl.DeviceIdType.MESH)
        pl.semaphore_wait(barrier, N_DEV - 1)

    with jax.named_scope("c_proj"):
        x2 = x2_ref[...]
        cf[:, pl.ds(me * DCS, DCS)] = jnp.dot(
            x2, wdkv_ref[...], preferred_element_type=F32).astype(BF16)
        wkg[pl.ds(me * DCS, DCS), :] = wuk_ref[:, pl.ds(me * GC, GC)]
        wvg[pl.ds(me * DCS, DCS), :] = wuv_ref[:, pl.ds(me * GC, GC)]

    gather = {}
    for k in (1, 3, 2):
        tgt = (me + k) % N_DEV
        base = 3 * (N_DEV - 1 - k)
        trio = []
        for t, (src, dst) in enumerate([
            (cf.at[:, pl.ds(me * DCS, DCS)],
             cf.at[:, pl.ds(me * DCS, DCS)]),
            (wuk_ref.at[:, pl.ds(tgt * GC, GC)],
             wkg.at[pl.ds(me * DCS, DCS), :]),
            (wuv_ref.at[:, pl.ds(tgt * GC, GC)],
             wvg.at[pl.ds(me * DCS, DCS), :]),
        ]):
            rdma = pltpu.make_async_remote_copy(
                src_ref=src, dst_ref=dst,
                send_sem=g_send.at[base + t], recv_sem=g_recv.at[base + t],
                device_id=(tgt,), device_id_type=pl.DeviceIdType.MESH,
            )
            rdma.start()
            trio.append(rdma)
        gather[k] = trio

    with jax.named_scope("q_proj"):
        qg[...] = jnp.dot(x2, wq_ref[...].astype(BF16),
                          preferred_element_type=F32).astype(BF16)
        qrg[...] = jnp.dot(x2, wqr_ref[...].astype(BF16),
                           preferred_element_type=F32).astype(BF16)
        krb[...] = jnp.dot(x2, wkr_ref[...],
                           preferred_element_type=F32).astype(BF16)

    scale = (Dh + Dr) ** -0.5
    cdims = (((1,), (1,)), ((), ()))
    with jax.named_scope("qr_scores"):
        for h in range(HG):
            for b in range(B):
                rows = pl.ds(b * S, S)
                scs[h * B + b] = lax.dot_general(
                    qrg[rows, pl.ds(h * Dr, Dr)], krb[rows, :], cdims,
                    preferred_element_type=F32).astype(BF16)

    with jax.named_scope("kv_own"):
        own_rows = pl.ds(me * DCS, DCS)
        kg[...] = jnp.dot(cf[:, own_rows], wkg[own_rows, :],
                          preferred_element_type=F32).astype(BF16)
        vg[...] = jnp.dot(cf[:, own_rows], wvg[own_rows, :],
                          preferred_element_type=F32).astype(BF16)
    for k in (1, 3, 2):
        with jax.named_scope(f"gather_wait#k={k}"):
            for rdma in gather[k]:
                rdma.wait()
        with jax.named_scope(f"kv_acc#k={k}"):
            p = (me + N_DEV - k) % N_DEV
            prows = pl.ds(p * DCS, DCS)
            kg[...] = (kg[...] + jnp.dot(
                cf[:, prows], wkg[prows, :],
                preferred_element_type=F32).astype(BF16))
            vg[...] = (vg[...] + jnp.dot(
                cf[:, prows], wvg[prows, :],
                preferred_element_type=F32).astype(BF16))

    srcs = [me, (me + 3) % N_DEV, (me + 1) % N_DEV, (me + 2) % N_DEV]

    def wo_dma(j):
        return pltpu.make_async_copy(
            wo_ref.at[pl.ds(srcs[j] * GC, GC), :], wos.at[j % 2],
            wo_sems.at[j % 2])

    dma0 = wo_dma(0)
    dma0.start()
    dma1 = wo_dma(1)
    dma1.start()

    ag = [[None] * HG for _ in range(N_DEV - 1)]
    for h in range(HG):
        cols = pl.ds(h * Dh, Dh)
        with jax.named_scope(f"attn#h={h}"):
            for b in range(B):
                rows = pl.ds(b * S, S)
                q = qg[rows, cols]
                k_ = kg[rows, cols]
                sc = lax.dot_general(q, k_, cdims,
                                     preferred_element_type=F32)
                sc = (sc + scs[h * B + b].astype(F32)) * scale
                m = jnp.max(sc, axis=1, keepdims=True)
                e = jnp.exp(sc - m)
                p = (e / jnp.sum(e, axis=1, keepdims=True)).astype(BF16)
                og[rows, cols] = jnp.dot(
                    p, vg[rows, cols],
                    preferred_element_type=F32).astype(BF16)
        for k in (1, 3, 2):
            tgt = (me + k) % N_DEV
            slot = N_DEV - 1 - k
            rdma = pltpu.make_async_remote_copy(
                src_ref=og.at[:, cols],
                dst_ref=obuf.at[slot, :, cols],
                send_sem=a_send.at[slot * HG + h],
                recv_sem=a_recv.at[slot * HG + h],
                device_id=(tgt,), device_id_type=pl.DeviceIdType.MESH,
            )
            rdma.start()
            ag[slot][h] = rdma

    with jax.named_scope("wo_own"):
        dma0.wait()
        out_ref[...] = jnp.dot(og[...], wos[0].astype(BF16),
                               preferred_element_type=F32)
        dma2 = wo_dma(2)
        dma2.start()
    with jax.named_scope("ag_wait#j=1"):
        for h in range(HG):
            ag[2][h].wait()
        dma1.wait()
    with jax.named_scope("wo_blk#j=1"):
        out_ref[...] += jnp.dot(obuf[2], wos[1].astype(BF16),
                                preferred_element_type=F32)
        dma3 = wo_dma(3)
        dma3.start()
    with jax.named_scope("ag_wait#j=2"):
        for h in range(HG):
            ag[0][h].wait()
        dma2.wait()
    with jax.named_scope("wo_blk#j=2"):
        out_ref[...] += jnp.dot(obuf[0], wos[0].astype(BF16),
                                preferred_element_type=F32)
    with jax.named_scope("ag_wait#j=3"):
        for h in range(HG):
            ag[1][h].wait()
        dma3.wait()
    with jax.named_scope("wo_blk#j=3"):
        out_ref[...] += jnp.dot(obuf[1], wos[1].astype(BF16),
                                preferred_element_type=F32)


def kernel(x, Wdkv, Wuk, Wuv, Wq, Wqr, Wkr, Wo):
    f = BF16
    me = lax.axis_index("i")
    x2 = x.astype(f).reshape(BS, D)
    Wq_g = lax.dynamic_slice(Wq, (0, me * GC), (D, GC))
    Wqr_g = lax.dynamic_slice(Wqr, (0, me * HG * Dr), (D, HG * Dr))
    out = pl.pallas_call(
        _mla_body,
        out_shape=jax.ShapeDtypeStruct((BS, D), F32),
        in_specs=[pl.BlockSpec(memory_space=pltpu.VMEM)] * 7
        + [pl.BlockSpec(memory_space=pltpu.MemorySpace.HBM)],
        out_specs=pl.BlockSpec(memory_space=pltpu.VMEM),
        scratch_shapes=[
            pltpu.VMEM((BS, N_DEV * DCS), f),
            pltpu.VMEM((N_DEV * DCS, GC), f),
            pltpu.VMEM((N_DEV * DCS, GC), f),
            pltpu.VMEM((BS, GC), f),
            pltpu.VMEM((BS, HG * Dr), f),
            pltpu.VMEM((BS, Dr), f),
            pltpu.VMEM((BS, GC), f),
            pltpu.VMEM((BS, GC), f),
            pltpu.VMEM((HG * B, S, S), f),
            pltpu.VMEM((BS, GC), f),
            pltpu.VMEM((N_DEV - 1, BS, GC), f),
            pltpu.VMEM((2, GC, D), F32),
            pltpu.SemaphoreType.DMA((3 * (N_DEV - 1),)),
            pltpu.SemaphoreType.DMA((3 * (N_DEV - 1),)),
            pltpu.SemaphoreType.DMA((HG * (N_DEV - 1),)),
            pltpu.SemaphoreType.DMA((HG * (N_DEV - 1),)),
            pltpu.SemaphoreType.DMA((2,)),
        ],
        compiler_params=pltpu.CompilerParams(collective_id=0),
    )(x2, Wdkv.astype(f), Wuk.astype(f), Wuv.astype(f), Wq_g, Wqr_g,
      Wkr.astype(f), Wo)
    return out.reshape(B, S, D)
